# baseline (speedup 1.0000x reference)
import functools

import jax
import jax.numpy as jnp
from jax import lax
from jax.experimental import pallas as pl
from jax.experimental.pallas import tpu as pltpu
from jax.experimental.pallas import tpu_sc as plsc

_D = 128
_B = 16384
_EPS = 1e-5

_NC = 2
_NS = 16
_NW = _NC * _NS
_B_SC = _B // 2
_B_PER_W = _B_SC // _NW


def _dense_body(emb_ref, w_ref, b_ref, g_ref, beta_ref, out_ref):
    emb = emb_ref[...]
    w = w_ref[...]
    h = lax.dot_general(emb, w, (((1,), (1,)), ((), ())),
                        preferred_element_type=jnp.float32)
    h = h + b_ref[...]
    mu = jnp.mean(h, axis=-1, keepdims=True)
    d = h - mu
    var = jnp.mean(d * d, axis=-1, keepdims=True)
    out_ref[...] = d * lax.rsqrt(var + _EPS) * g_ref[...] + beta_ref[...]


_dense_call = pl.pallas_call(
    _dense_body,
    out_shape=jax.ShapeDtypeStruct((2, _D), jnp.float32),
)


_L = 16
_NCG = _D // _L
_NCHUNK = 2
_CH = _B_PER_W // _NCHUNK


def _gather_body(table_hbm, idx_hbm, out_hbm, table_v, idx_v, rows_v, sem):
    wid = lax.axis_index("s") * _NC + lax.axis_index("c")
    base = wid * _B_PER_W
    pltpu.sync_copy(table_hbm, table_v)
    pltpu.sync_copy(idx_hbm.at[pl.ds(base, _B_PER_W)], idx_v)
    r0 = [table_v[0, pl.ds(_L * c, _L)] for c in range(_NCG)]
    dl = [table_v[1, pl.ds(_L * c, _L)] - r0[c] for c in range(_NCG)]

    copies = []
    for ch in range(_NCHUNK):
        def body(j, carry, ch=ch):
            xv = idx_v[pl.ds(ch * _CH + j * _L, _L)].astype(jnp.float32)
            for l in range(_L):
                xf = jnp.full((_L,), xv[l], jnp.float32)
                i = ch * _CH + j * _L + l
                for c in range(_NCG):
                    rows_v[i, pl.ds(_L * c, _L)] = r0[c] + xf * dl[c]
            return carry

        lax.fori_loop(0, _CH // _L, body, 0)
        copies.append(pltpu.async_copy(
            rows_v.at[pl.ds(ch * _CH, _CH)],
            out_hbm.at[pl.ds(base + ch * _CH, _CH)], sem))
    for c in copies:
        c.wait()


@functools.cache
def _gather_call():
    return pl.kernel(
        _gather_body,
        out_type=jax.ShapeDtypeStruct((_B_SC, _D), jnp.float32),
        mesh=plsc.VectorSubcoreMesh(core_axis_name="c", subcore_axis_name="s",
                                    num_cores=_NC, num_subcores=_NS),
        scratch_types=[
            pltpu.VMEM((2, _D), jnp.float32),
            pltpu.VMEM((_B_PER_W,), jnp.int32),
            pltpu.VMEM((_B_PER_W, _D), jnp.float32),
            pltpu.SemaphoreType.DMA,
        ],
    )


_BT = 1024


def _select_body(x_ref, t_ref, o_ref):
    xf = x_ref[0]
    t0 = t_ref[0:1, :]
    d = t_ref[1:2, :] - t0
    o_ref[...] = t0 + xf * d


def _select_call(nrows):
    grid = nrows // _BT
    return pl.pallas_call(
        _select_body,
        grid=(grid,),
        in_specs=[
            pl.BlockSpec((1, _BT, 1), lambda i: (i, 0, 0)),
            pl.BlockSpec((2, _D), lambda i: (0, 0)),
        ],
        out_specs=pl.BlockSpec((_BT, _D), lambda i: (i, 0)),
        out_shape=jax.ShapeDtypeStruct((nrows, _D), jnp.float32),
    )


def kernel(x, emb_table, W, b, gamma, beta):
    idx = x.astype(jnp.int32)
    table = _dense_call(emb_table, W, b.reshape(1, _D),
                        gamma.reshape(1, _D), beta.reshape(1, _D))
    sc_out = _gather_call()(table, idx[:_B_SC])
    xf = idx[_B_SC:].astype(jnp.float32).reshape((_B - _B_SC) // _BT, _BT, 1)
    tc_out = _select_call(_B - _B_SC)(xf, table)
    return jnp.concatenate([sc_out, tc_out], axis=0)


# speedup vs baseline: 1.1693x; 1.1693x over previous
import functools

import jax
import jax.numpy as jnp
from jax import lax
from jax.experimental import pallas as pl
from jax.experimental.pallas import tpu as pltpu
from jax.experimental.pallas import tpu_sc as plsc

_D = 128
_B = 16384
_EPS = 1e-5

_NC = 2
_NS = 16
_NW = _NC * _NS
_B_SC = _B // 2
_B_PER_W = _B_SC // _NW


def _dense_body(emb_ref, w_ref, b_ref, g_ref, beta_ref, out_ref):
    emb = emb_ref[...]
    w = w_ref[...]
    h = lax.dot_general(emb, w, (((1,), (1,)), ((), ())),
                        preferred_element_type=jnp.float32)
    h = h + b_ref[...]
    mu = jnp.mean(h, axis=-1, keepdims=True)
    d = h - mu
    var = jnp.mean(d * d, axis=-1, keepdims=True)
    out_ref[...] = d * lax.rsqrt(var + _EPS) * g_ref[...] + beta_ref[...]


_dense_call = pl.pallas_call(
    _dense_body,
    out_shape=jax.ShapeDtypeStruct((2, _D), jnp.float32),
)


_L = 16
_NCG = _D // _L
_NCHUNK = 2
_CH = _B_PER_W // _NCHUNK


def _gather_body(table_hbm, idx_hbm, out_hbm, table_v, idx_v, rows_v, sem):
    wid = lax.axis_index("s") * _NC + lax.axis_index("c")
    base = wid * _B_PER_W
    pltpu.sync_copy(table_hbm, table_v)
    pltpu.sync_copy(idx_hbm.at[pl.ds(base, _B_PER_W)], idx_v)
    r0 = [table_v[0, pl.ds(_L * c, _L)] for c in range(_NCG)]
    dl = [table_v[1, pl.ds(_L * c, _L)] - r0[c] for c in range(_NCG)]

    copies = []
    for ch in range(_NCHUNK):
        def body(j, carry, ch=ch):
            xv = idx_v[pl.ds(ch * _CH + j * _L, _L)].astype(jnp.float32)
            for l in range(_L):
                xf = jnp.full((_L,), xv[l], jnp.float32)
                i = ch * _CH + j * _L + l
                for c in range(_NCG):
                    rows_v[i, pl.ds(_L * c, _L)] = r0[c] + xf * dl[c]
            return carry

        lax.fori_loop(0, _CH // _L, body, 0)
        copies.append(pltpu.async_copy(
            rows_v.at[pl.ds(ch * _CH, _CH)],
            out_hbm.at[pl.ds(base + ch * _CH, _CH)], sem))
    for c in copies:
        c.wait()


@functools.cache
def _gather_call():
    return pl.kernel(
        _gather_body,
        out_type=jax.ShapeDtypeStruct((_B, _D), jnp.float32),
        mesh=plsc.VectorSubcoreMesh(core_axis_name="c", subcore_axis_name="s",
                                    num_cores=_NC, num_subcores=_NS),
        scratch_types=[
            pltpu.VMEM((2, _D), jnp.float32),
            pltpu.VMEM((_B_PER_W,), jnp.int32),
            pltpu.VMEM((_B_PER_W, _D), jnp.float32),
            pltpu.SemaphoreType.DMA,
        ],
    )


_BT = 1024


def _select_body(x_ref, t_ref, buf_ref, o_ref):
    del buf_ref
    xf = x_ref[0]
    t0 = t_ref[0:1, :]
    d = t_ref[1:2, :] - t0
    o_ref[...] = t0 + xf * d


def _select_call():
    return pl.pallas_call(
        _select_body,
        grid=((_B - _B_SC) // _BT,),
        in_specs=[
            pl.BlockSpec((1, _BT, 1), lambda i: (i, 0, 0)),
            pl.BlockSpec((2, _D), lambda i: (0, 0)),
            pl.BlockSpec(memory_space=pltpu.MemorySpace.HBM),
        ],
        out_specs=pl.BlockSpec((_BT, _D), lambda i: (i + _B_SC // _BT, 0)),
        out_shape=jax.ShapeDtypeStruct((_B, _D), jnp.float32),
        input_output_aliases={2: 0},
    )


def kernel(x, emb_table, W, b, gamma, beta):
    idx = x.astype(jnp.int32)
    table = _dense_call(emb_table, W, b.reshape(1, _D),
                        gamma.reshape(1, _D), beta.reshape(1, _D))
    sc_out = _gather_call()(table, idx[:_B_SC])
    xf = idx[_B_SC:].astype(jnp.float32).reshape((_B - _B_SC) // _BT, _BT, 1)
    return _select_call()(xf, table, sc_out)


# speedup vs baseline: 1.8677x; 1.5973x over previous
import functools

import jax
import jax.numpy as jnp
from jax import lax
from jax.experimental import pallas as pl
from jax.experimental.pallas import tpu as pltpu
from jax.experimental.pallas import tpu_sc as plsc

_D = 128
_B = 16384
_EPS = 1e-5

_NC = 2
_NS = 16
_NW = _NC * _NS
_B_PER_W = _B // _NW


def _dense_body(emb_ref, w_ref, b_ref, g_ref, beta_ref, out_ref):
    emb = emb_ref[...]
    w = w_ref[...]
    h = lax.dot_general(emb, w, (((1,), (1,)), ((), ())),
                        preferred_element_type=jnp.float32)
    h = h + b_ref[...]
    mu = jnp.mean(h, axis=-1, keepdims=True)
    d = h - mu
    var = jnp.mean(d * d, axis=-1, keepdims=True)
    out_ref[...] = d * lax.rsqrt(var + _EPS) * g_ref[...] + beta_ref[...]


_dense_call = pl.pallas_call(
    _dense_body,
    out_shape=jax.ShapeDtypeStruct((2, _D), jnp.float32),
)


_L = 16
_NCG = _D // _L
_NCHUNK = 2
_CH = _B_PER_W // _NCHUNK


_KG = 4
_NCOMBO = 1 << _KG


def _gather_body(table_hbm, idx_hbm, out_hbm, table_v, idx_v, combo_v, sem):
    wid = lax.axis_index("s") * _NC + lax.axis_index("c")
    base = wid * _B_PER_W
    pltpu.sync_copy(table_hbm, table_v)
    pltpu.sync_copy(idx_hbm.at[pl.ds(base, _B_PER_W)], idx_v)
    r0 = [table_v[0, pl.ds(_L * c, _L)] for c in range(_NCG)]
    r1 = [table_v[1, pl.ds(_L * c, _L)] for c in range(_NCG)]

    for m in range(_NCOMBO):
        for pos in range(_KG):
            src = r1 if (m >> pos) & 1 else r0
            for c in range(_NCG):
                combo_v[m * _KG + pos, pl.ds(_L * c, _L)] = src[c]

    def gbody(q, carry):
        v = idx_v[pl.ds(q * _L, _L)]
        for a in range(_L // _KG):
            key = (v[_KG * a] + v[_KG * a + 1] * 2
                   + v[_KG * a + 2] * 4 + v[_KG * a + 3] * 8)
            off = q * _L + a * _KG
            pltpu.async_copy(combo_v.at[pl.ds(key * _KG, _KG)],
                             out_hbm.at[pl.ds(base + off, _KG)], sem)
        return carry

    lax.fori_loop(0, _B_PER_W // _L, gbody, 0)
    for _ in range(_B_PER_W // (_KG * _NCOMBO)):
        pltpu.make_async_copy(out_hbm.at[pl.ds(base, _KG * _NCOMBO)],
                              combo_v, sem).wait()


@functools.cache
def _gather_call():
    return pl.kernel(
        _gather_body,
        out_type=jax.ShapeDtypeStruct((_B, _D), jnp.float32),
        mesh=plsc.VectorSubcoreMesh(core_axis_name="c", subcore_axis_name="s",
                                    num_cores=_NC, num_subcores=_NS),
        scratch_types=[
            pltpu.VMEM((2, _D), jnp.float32),
            pltpu.VMEM((_B_PER_W,), jnp.int32),
            pltpu.VMEM((_KG * _NCOMBO, _D), jnp.float32),
            pltpu.SemaphoreType.DMA,
        ],
    )


def kernel(x, emb_table, W, b, gamma, beta):
    idx = x.astype(jnp.int32)
    table = _dense_call(emb_table, W, b.reshape(1, _D),
                        gamma.reshape(1, _D), beta.reshape(1, _D))
    return _gather_call()(table, idx)


# speedup vs baseline: 1.8864x; 1.0100x over previous
import functools

import jax
import jax.numpy as jnp
from jax import lax
from jax.experimental import pallas as pl
from jax.experimental.pallas import tpu as pltpu
from jax.experimental.pallas import tpu_sc as plsc

_D = 128
_B = 16384
_EPS = 1e-5

_NC = 2
_NS = 16
_NW = _NC * _NS
_B_PER_W = _B // _NW


def _dense_body(emb_ref, w_ref, b_ref, g_ref, beta_ref, out_ref):
    emb = emb_ref[...]
    w = w_ref[...]
    h = lax.dot_general(emb, w, (((1,), (1,)), ((), ())),
                        preferred_element_type=jnp.float32)
    h = h + b_ref[...]
    mu = jnp.mean(h, axis=-1, keepdims=True)
    d = h - mu
    var = jnp.mean(d * d, axis=-1, keepdims=True)
    out_ref[...] = d * lax.rsqrt(var + _EPS) * g_ref[...] + beta_ref[...]


_dense_call = pl.pallas_call(
    _dense_body,
    out_shape=jax.ShapeDtypeStruct((2, _D), jnp.float32),
)


_L = 16
_NCG = _D // _L
_NCHUNK = 2
_CH = _B_PER_W // _NCHUNK


_KG = 4
_NCOMBO = 1 << _KG


def _gather_body(table_hbm, idx_hbm, out_hbm, table_v, idx_v, combo_v, sem,
                 idx_sem):
    wid = lax.axis_index("s") * _NC + lax.axis_index("c")
    base = wid * _B_PER_W
    idx_copy = pltpu.async_copy(idx_hbm.at[pl.ds(base, _B_PER_W)], idx_v,
                                idx_sem)
    pltpu.sync_copy(table_hbm, table_v)
    r0 = [table_v[0, pl.ds(_L * c, _L)] for c in range(_NCG)]
    r1 = [table_v[1, pl.ds(_L * c, _L)] for c in range(_NCG)]

    for m in range(_NCOMBO):
        for pos in range(_KG):
            src = r1 if (m >> pos) & 1 else r0
            for c in range(_NCG):
                combo_v[m * _KG + pos, pl.ds(_L * c, _L)] = src[c]
    idx_copy.wait()

    def gbody(q, carry):
        v = idx_v[pl.ds(q * _L, _L)]
        for a in range(_L // _KG):
            key = (v[_KG * a] + v[_KG * a + 1] * 2
                   + v[_KG * a + 2] * 4 + v[_KG * a + 3] * 8)
            off = q * _L + a * _KG
            pltpu.async_copy(combo_v.at[pl.ds(key * _KG, _KG)],
                             out_hbm.at[pl.ds(base + off, _KG)], sem)
        return carry

    lax.fori_loop(0, _B_PER_W // _L, gbody, 0)
    for _ in range(_B_PER_W // (_KG * _NCOMBO)):
        pltpu.make_async_copy(out_hbm.at[pl.ds(base, _KG * _NCOMBO)],
                              combo_v, sem).wait()


@functools.cache
def _gather_call():
    return pl.kernel(
        _gather_body,
        out_type=jax.ShapeDtypeStruct((_B, _D), jnp.float32),
        mesh=plsc.VectorSubcoreMesh(core_axis_name="c", subcore_axis_name="s",
                                    num_cores=_NC, num_subcores=_NS),
        scratch_types=[
            pltpu.VMEM((2, _D), jnp.float32),
            pltpu.VMEM((_B_PER_W,), jnp.int32),
            pltpu.VMEM((_KG * _NCOMBO, _D), jnp.float32),
            pltpu.SemaphoreType.DMA,
            pltpu.SemaphoreType.DMA,
        ],
    )


def kernel(x, emb_table, W, b, gamma, beta):
    idx = x.astype(jnp.int32)
    table = _dense_call(emb_table, W, b.reshape(1, _D),
                        gamma.reshape(1, _D), beta.reshape(1, _D))
    return _gather_call()(table, idx)
